# baseline (speedup 1.0000x reference)
import functools

import jax
import jax.numpy as jnp
from jax import lax
from jax.experimental import pallas as pl
from jax.experimental.pallas import tpu as pltpu
from jax.experimental.pallas import tpu_sc as plsc

N = 10000
E = 320000
D = 128

NC = 2
NS = 16
NW = NC * NS

NP = 10240
K = 160
NSLOT = NP * K

EPT = E // NW
CB = 80
NCHUNK = EPT // CB

SPT = NSLOT // NW
CB2 = 128
NCHUNK2 = SPT // CB2

EBLK = 512
EGRID = E // EBLK
VB = 64
VGRID = NP // VB

_mesh = plsc.VectorSubcoreMesh(core_axis_name="c", subcore_axis_name="s")


def _wid():
    return lax.axis_index("s") * NC + lax.axis_index("c")


@functools.partial(
    pl.kernel,
    mesh=_mesh,
    out_type=(
        jax.ShapeDtypeStruct((E, D), jnp.float32),
        jax.ShapeDtypeStruct((E, D), jnp.float32),
    ),
    scratch_types=[
        pltpu.VMEM((CB,), jnp.int32),
        pltpu.VMEM((CB,), jnp.int32),
        pltpu.VMEM((CB, D), jnp.float32),
        pltpu.VMEM((CB, D), jnp.float32),
        pltpu.SemaphoreType.DMA,
        pltpu.SemaphoreType.DMA,
    ],
)
def _sc_gather2(ei0_hbm, ei1_hbm, xv_hbm, s0_hbm, s1_hbm,
                idx0, idx1, bufa, bufb, sema, semb):
    base = pl.multiple_of(_wid() * EPT, 8)

    def chunk(i, carry):
        cs = pl.multiple_of(base + i * CB, 8)
        pltpu.sync_copy(ei0_hbm.at[pl.ds(cs, CB)], idx0)
        pltpu.sync_copy(ei1_hbm.at[pl.ds(cs, CB)], idx1)
        ca = pltpu.async_copy(xv_hbm.at[idx0], bufa, sema)
        cb = pltpu.async_copy(xv_hbm.at[idx1], bufb, semb)
        ca.wait()
        cb.wait()
        pltpu.sync_copy(bufa, s0_hbm.at[pl.ds(cs, CB), :])
        pltpu.sync_copy(bufb, s1_hbm.at[pl.ds(cs, CB), :])
        return carry

    lax.fori_loop(0, NCHUNK, chunk, 0)


@functools.partial(
    pl.kernel,
    mesh=_mesh,
    out_type=jax.ShapeDtypeStruct((NSLOT, D), jnp.float32),
    scratch_types=[
        pltpu.VMEM((CB2,), jnp.int32),
        pltpu.VMEM((CB2, D), jnp.float32),
        pltpu.SemaphoreType.DMA,
    ],
)
def _sc_gather1(slot_hbm, xe_hbm, gath_hbm, idx, buf, sem):
    base = pl.multiple_of(_wid() * SPT, 8)

    def chunk(i, carry):
        cs = pl.multiple_of(base + i * CB2, 8)
        pltpu.sync_copy(slot_hbm.at[pl.ds(cs, CB2)], idx)
        pltpu.async_copy(xe_hbm.at[idx], buf, sem).wait()
        pltpu.sync_copy(buf, gath_hbm.at[pl.ds(cs, CB2), :])
        return carry

    lax.fori_loop(0, NCHUNK2, chunk, 0)


def _edge_body(s0_ref, s1_ref, xe_ref, w_ref, b_ref, out_ref):
    s0 = s0_ref[...]
    s1 = s1_ref[...]
    mean = 0.5 * (s0 + s1)
    mx = jnp.maximum(s0, s1)
    sq = 0.5 * (s0 * s0 + s1 * s1)
    hp = lax.Precision.HIGHEST
    h = jnp.dot(mean, w_ref[0:D, :], preferred_element_type=jnp.float32,
                precision=hp)
    h = h + jnp.dot(mx, w_ref[D:2 * D, :], preferred_element_type=jnp.float32,
                    precision=hp)
    h = h + jnp.dot(sq, w_ref[2 * D:3 * D, :],
                    preferred_element_type=jnp.float32, precision=hp)
    out_ref[...] = xe_ref[...] + jnp.maximum(h + b_ref[...], 0.0)


_edge_call = pl.pallas_call(
    _edge_body,
    grid=(EGRID,),
    in_specs=[
        pl.BlockSpec((EBLK, D), lambda i: (i, 0)),
        pl.BlockSpec((EBLK, D), lambda i: (i, 0)),
        pl.BlockSpec((EBLK, D), lambda i: (i, 0)),
        pl.BlockSpec((3 * D, D), lambda i: (0, 0)),
        pl.BlockSpec((1, D), lambda i: (0, 0)),
    ],
    out_specs=pl.BlockSpec((EBLK, D), lambda i: (i, 0)),
    out_shape=jax.ShapeDtypeStruct((E, D), jnp.float32),
)


def _vert_body(gath_ref, m_ref, sp_ref, ep_ref, xv_ref, w_ref, b_ref, out_ref):
    g = gath_ref[...]
    m = m_ref[...]
    gs = g * m
    gm = gs + (m - 1.0) * 1e30
    s = jnp.sum(gs.reshape(VB, K, D), axis=1)
    mx = jnp.max(gm.reshape(VB, K, D), axis=1)
    sq = jnp.sum((gs * g).reshape(VB, K, D), axis=1)
    deg = (ep_ref[...] - sp_ref[...]).astype(jnp.float32)
    inv = 1.0 / jnp.maximum(deg, 1.0)
    mean = s * inv
    mx = jnp.where(deg > 0.0, mx, 0.0)
    sqm = sq * inv
    hp = lax.Precision.HIGHEST
    h = jnp.dot(mean, w_ref[0:D, :], preferred_element_type=jnp.float32,
                precision=hp)
    h = h + jnp.dot(mx, w_ref[D:2 * D, :], preferred_element_type=jnp.float32,
                    precision=hp)
    h = h + jnp.dot(sqm, w_ref[2 * D:3 * D, :],
                    preferred_element_type=jnp.float32, precision=hp)
    out_ref[...] = xv_ref[...] + jnp.maximum(h + b_ref[...], 0.0)


_vert_call = pl.pallas_call(
    _vert_body,
    grid=(VGRID,),
    in_specs=[
        pl.BlockSpec((VB * K, D), lambda i: (i, 0)),
        pl.BlockSpec((VB * K, 1), lambda i: (i, 0)),
        pl.BlockSpec((VB, 1), lambda i: (i, 0)),
        pl.BlockSpec((VB, 1), lambda i: (i, 0)),
        pl.BlockSpec((VB, D), lambda i: (i, 0)),
        pl.BlockSpec((3 * D, D), lambda i: (0, 0)),
        pl.BlockSpec((1, D), lambda i: (0, 0)),
    ],
    out_specs=pl.BlockSpec((VB, D), lambda i: (i, 0)),
    out_shape=jax.ShapeDtypeStruct((NP, D), jnp.float32),
)


def kernel(edge_index, xe, xv, line_W, line_b, linv_W, linv_b):
    ei0 = edge_index[0]
    ei1 = edge_index[1]

    endpoints = jnp.concatenate([ei0, ei1])
    order = jnp.argsort(endpoints).astype(jnp.int32)
    eid = jnp.where(order >= E, order - E, order)
    sorted_rows = jnp.take(endpoints, order)
    rowptr = jnp.searchsorted(
        sorted_rows, jnp.arange(NP + 1, dtype=jnp.int32)).astype(jnp.int32)
    sp = rowptr[:NP, None]
    ep = rowptr[1:NP + 1, None]
    deg = ep - sp
    pos = sp + jnp.arange(K, dtype=jnp.int32)[None, :]
    valid = jnp.arange(K, dtype=jnp.int32)[None, :] < deg
    slotpad = jnp.where(
        valid, jnp.take(eid, jnp.minimum(pos, 2 * E - 1)), 0
    ).reshape(NSLOT).astype(jnp.int32)
    vmask = valid.astype(jnp.float32).reshape(NSLOT, 1)

    xvp = jnp.concatenate([xv, jnp.zeros((NP - N, D), jnp.float32)])
    xeb = xe
    b_line = line_b.reshape(1, D)
    b_linv = linv_b.reshape(1, D)

    for _ in range(2):
        s0, s1 = _sc_gather2(ei0, ei1, xvp)
        xeb = _edge_call(s0, s1, xeb, line_W, b_line)
        gath = _sc_gather1(slotpad, xeb)
        xvp = _vert_call(gath, vmask, sp, ep, xvp, linv_W, b_linv)

    return (xeb, xvp[:N])


# speedup vs baseline: 6.4374x; 6.4374x over previous
import functools

import jax
import jax.numpy as jnp
from jax import lax
from jax.experimental import pallas as pl
from jax.experimental.pallas import tpu as pltpu
from jax.experimental.pallas import tpu_sc as plsc

N = 10000
E = 320000
D = 128

NC = 2
NS = 16
NW = NC * NS

NP = 10240
K = 160
NSLOT = NP * K

EPT = E // NW
CB = 80
NCHUNK = EPT // CB

EBLK = 512
EGRID = E // EBLK
VB = 64
VGRID = NP // VB

_mesh = plsc.VectorSubcoreMesh(core_axis_name="c", subcore_axis_name="s")


def _wid():
    return lax.axis_index("s") * NC + lax.axis_index("c")


@functools.partial(
    pl.kernel,
    mesh=_mesh,
    out_type=(
        jax.ShapeDtypeStruct((E, D), jnp.float32),
        jax.ShapeDtypeStruct((E, D), jnp.float32),
    ),
    scratch_types=[
        pltpu.VMEM((CB,), jnp.int32),
        pltpu.VMEM((CB,), jnp.int32),
        pltpu.VMEM((CB, D), jnp.float32),
        pltpu.VMEM((CB, D), jnp.float32),
        pltpu.SemaphoreType.DMA,
        pltpu.SemaphoreType.DMA,
    ],
)
def _sc_gather2(ei0_hbm, ei1_hbm, xv_hbm, s0_hbm, s1_hbm,
                idx0, idx1, bufa, bufb, sema, semb):
    base = pl.multiple_of(_wid() * EPT, 8)

    def chunk(i, carry):
        cs = pl.multiple_of(base + i * CB, 8)
        pltpu.sync_copy(ei0_hbm.at[pl.ds(cs, CB)], idx0)
        pltpu.sync_copy(ei1_hbm.at[pl.ds(cs, CB)], idx1)
        ca = pltpu.async_copy(xv_hbm.at[idx0], bufa, sema)
        cb = pltpu.async_copy(xv_hbm.at[idx1], bufb, semb)
        ca.wait()
        cb.wait()
        pltpu.sync_copy(bufa, s0_hbm.at[pl.ds(cs, CB), :])
        pltpu.sync_copy(bufb, s1_hbm.at[pl.ds(cs, CB), :])
        return carry

    lax.fori_loop(0, NCHUNK, chunk, 0)


@functools.partial(
    pl.kernel,
    mesh=_mesh,
    out_type=jax.ShapeDtypeStruct((NSLOT + 8, D), jnp.float32),
    scratch_types=[
        pltpu.VMEM((CB,), jnp.int32),
        pltpu.VMEM((CB,), jnp.int32),
        pltpu.VMEM((CB, D), jnp.float32),
        pltpu.SemaphoreType.DMA,
        pltpu.SemaphoreType.DMA,
    ],
)
def _sc_scatter(sc0_hbm, sc1_hbm, xe_hbm, gath_hbm, idx0, idx1, buf,
                sema, semb):
    base = pl.multiple_of(_wid() * EPT, 8)

    def chunk(i, carry):
        cs = pl.multiple_of(base + i * CB, 8)
        pltpu.sync_copy(sc0_hbm.at[pl.ds(cs, CB)], idx0)
        pltpu.sync_copy(sc1_hbm.at[pl.ds(cs, CB)], idx1)
        pltpu.sync_copy(xe_hbm.at[pl.ds(cs, CB), :], buf)
        ca = pltpu.async_copy(buf, gath_hbm.at[idx0], sema)
        cb = pltpu.async_copy(buf, gath_hbm.at[idx1], semb)
        ca.wait()
        cb.wait()
        return carry

    lax.fori_loop(0, NCHUNK, chunk, 0)


def _edge_body(s0_ref, s1_ref, xe_ref, w_ref, b_ref, out_ref):
    s0 = s0_ref[...]
    s1 = s1_ref[...]
    mean = 0.5 * (s0 + s1)
    mx = jnp.maximum(s0, s1)
    sq = 0.5 * (s0 * s0 + s1 * s1)
    hp = lax.Precision.HIGHEST
    h = jnp.dot(mean, w_ref[0:D, :], preferred_element_type=jnp.float32,
                precision=hp)
    h = h + jnp.dot(mx, w_ref[D:2 * D, :], preferred_element_type=jnp.float32,
                    precision=hp)
    h = h + jnp.dot(sq, w_ref[2 * D:3 * D, :],
                    preferred_element_type=jnp.float32, precision=hp)
    out_ref[...] = xe_ref[...] + jnp.maximum(h + b_ref[...], 0.0)


_edge_call = pl.pallas_call(
    _edge_body,
    grid=(EGRID,),
    in_specs=[
        pl.BlockSpec((EBLK, D), lambda i: (i, 0)),
        pl.BlockSpec((EBLK, D), lambda i: (i, 0)),
        pl.BlockSpec((EBLK, D), lambda i: (i, 0)),
        pl.BlockSpec((3 * D, D), lambda i: (0, 0)),
        pl.BlockSpec((1, D), lambda i: (0, 0)),
    ],
    out_specs=pl.BlockSpec((EBLK, D), lambda i: (i, 0)),
    out_shape=jax.ShapeDtypeStruct((E, D), jnp.float32),
)


def _vert_body(gath_ref, m_ref, sp_ref, ep_ref, xv_ref, w_ref, b_ref, out_ref):
    g = gath_ref[...]
    msk = m_ref[...] > 0.5
    gs = jnp.where(msk, g, 0.0)
    gm = jnp.where(msk, g, -1e30)
    s = jnp.sum(gs.reshape(VB, K, D), axis=1)
    mx = jnp.max(gm.reshape(VB, K, D), axis=1)
    sq = jnp.sum((gs * gs).reshape(VB, K, D), axis=1)
    deg = (ep_ref[...] - sp_ref[...]).astype(jnp.float32)
    inv = 1.0 / jnp.maximum(deg, 1.0)
    mean = s * inv
    mx = jnp.where(deg > 0.0, mx, 0.0)
    sqm = sq * inv
    hp = lax.Precision.HIGHEST
    h = jnp.dot(mean, w_ref[0:D, :], preferred_element_type=jnp.float32,
                precision=hp)
    h = h + jnp.dot(mx, w_ref[D:2 * D, :], preferred_element_type=jnp.float32,
                    precision=hp)
    h = h + jnp.dot(sqm, w_ref[2 * D:3 * D, :],
                    preferred_element_type=jnp.float32, precision=hp)
    out_ref[...] = xv_ref[...] + jnp.maximum(h + b_ref[...], 0.0)


_vert_call = pl.pallas_call(
    _vert_body,
    grid=(VGRID,),
    in_specs=[
        pl.BlockSpec((VB * K, D), lambda i: (i, 0)),
        pl.BlockSpec((VB * K, 1), lambda i: (i, 0)),
        pl.BlockSpec((VB, 1), lambda i: (i, 0)),
        pl.BlockSpec((VB, 1), lambda i: (i, 0)),
        pl.BlockSpec((VB, D), lambda i: (i, 0)),
        pl.BlockSpec((3 * D, D), lambda i: (0, 0)),
        pl.BlockSpec((1, D), lambda i: (0, 0)),
    ],
    out_specs=pl.BlockSpec((VB, D), lambda i: (i, 0)),
    out_shape=jax.ShapeDtypeStruct((NP, D), jnp.float32),
)


def kernel(edge_index, xe, xv, line_W, line_b, linv_W, linv_b):
    ei0 = edge_index[0]
    ei1 = edge_index[1]

    endpoints = jnp.concatenate([ei0, ei1])
    order = jnp.argsort(endpoints).astype(jnp.int32)
    sorted_rows = jnp.take(endpoints, order)
    rowptr = jnp.searchsorted(
        sorted_rows, jnp.arange(NP + 1, dtype=jnp.int32)).astype(jnp.int32)
    sp = rowptr[:NP, None]
    ep = rowptr[1:NP + 1, None]
    deg = ep - sp
    valid = jnp.arange(K, dtype=jnp.int32)[None, :] < deg
    vmask = valid.astype(jnp.float32).reshape(NSLOT, 1)
    off_in_seg = jnp.arange(2 * E, dtype=jnp.int32) - jnp.take(rowptr,
                                                               sorted_rows)
    posj = jnp.where(off_in_seg < K,
                     sorted_rows * K + off_in_seg, NSLOT).astype(jnp.int32)
    scfull = jnp.zeros((2 * E,), jnp.int32).at[order].set(posj)
    sc0 = scfull[:E]
    sc1 = scfull[E:]

    xvp = jnp.concatenate([xv, jnp.zeros((NP - N, D), jnp.float32)])
    xeb = xe
    b_line = line_b.reshape(1, D)
    b_linv = linv_b.reshape(1, D)

    for _ in range(2):
        s0, s1 = _sc_gather2(ei0, ei1, xvp)
        xeb = _edge_call(s0, s1, xeb, line_W, b_line)
        gath = _sc_scatter(sc0, sc1, xeb)
        xvp = _vert_call(gath, vmask, sp, ep, xvp, linv_W, b_linv)

    return (xeb, xvp[:N])


# speedup vs baseline: 11.2120x; 1.7417x over previous
import functools

import jax
import jax.numpy as jnp
from jax import lax
from jax.experimental import pallas as pl
from jax.experimental.pallas import tpu as pltpu
from jax.experimental.pallas import tpu_sc as plsc

N = 10000
E = 320000
D = 128

NC = 2
NS = 16
NW = NC * NS

NP = 10240
K = 160
NSLOT = NP * K

EPT = E // NW
CB = 80
NCHUNK = EPT // CB

EBLK = 512
EGRID = E // EBLK
VB = 64
VGRID = NP // VB

_mesh = plsc.VectorSubcoreMesh(core_axis_name="c", subcore_axis_name="s")


def _wid():
    return lax.axis_index("s") * NC + lax.axis_index("c")


@functools.partial(
    pl.kernel,
    mesh=_mesh,
    out_type=(
        jax.ShapeDtypeStruct((E, D), jnp.float32),
        jax.ShapeDtypeStruct((E, D), jnp.float32),
    ),
    scratch_types=[
        pltpu.VMEM((CB,), jnp.int32),
        pltpu.VMEM((CB,), jnp.int32),
        pltpu.VMEM((CB, D), jnp.float32),
        pltpu.VMEM((CB, D), jnp.float32),
        pltpu.SemaphoreType.DMA,
        pltpu.SemaphoreType.DMA,
    ],
)
def _sc_gather2(ei0_hbm, ei1_hbm, xv_hbm, s0_hbm, s1_hbm,
                idx0, idx1, bufa, bufb, sema, semb):
    base = pl.multiple_of(_wid() * EPT, 8)

    def chunk(i, carry):
        cs = pl.multiple_of(base + i * CB, 8)
        pltpu.sync_copy(ei0_hbm.at[pl.ds(cs, CB)], idx0)
        pltpu.sync_copy(ei1_hbm.at[pl.ds(cs, CB)], idx1)
        ca = pltpu.async_copy(xv_hbm.at[idx0], bufa, sema)
        cb = pltpu.async_copy(xv_hbm.at[idx1], bufb, semb)
        ca.wait()
        cb.wait()
        pltpu.sync_copy(bufa, s0_hbm.at[pl.ds(cs, CB), :])
        pltpu.sync_copy(bufb, s1_hbm.at[pl.ds(cs, CB), :])
        return carry

    lax.fori_loop(0, NCHUNK, chunk, 0)


@functools.partial(
    pl.kernel,
    mesh=_mesh,
    out_type=jax.ShapeDtypeStruct((NSLOT + 8, D), jnp.float32),
    scratch_types=[
        pltpu.VMEM((CB,), jnp.int32),
        pltpu.VMEM((CB,), jnp.int32),
        pltpu.VMEM((CB, D), jnp.float32),
        pltpu.SemaphoreType.DMA,
        pltpu.SemaphoreType.DMA,
    ],
)
def _sc_scatter(sc0_hbm, sc1_hbm, xe_hbm, gath_hbm, idx0, idx1, buf,
                sema, semb):
    base = pl.multiple_of(_wid() * EPT, 8)

    def chunk(i, carry):
        cs = pl.multiple_of(base + i * CB, 8)
        pltpu.sync_copy(sc0_hbm.at[pl.ds(cs, CB)], idx0)
        pltpu.sync_copy(sc1_hbm.at[pl.ds(cs, CB)], idx1)
        pltpu.sync_copy(xe_hbm.at[pl.ds(cs, CB), :], buf)
        ca = pltpu.async_copy(buf, gath_hbm.at[idx0], sema)
        cb = pltpu.async_copy(buf, gath_hbm.at[idx1], semb)
        ca.wait()
        cb.wait()
        return carry

    lax.fori_loop(0, NCHUNK, chunk, 0)


def _edge_body(s0_ref, s1_ref, xe_ref, w_ref, b_ref, out_ref):
    s0 = s0_ref[...]
    s1 = s1_ref[...]
    mean = 0.5 * (s0 + s1)
    mx = jnp.maximum(s0, s1)
    sq = 0.5 * (s0 * s0 + s1 * s1)
    hp = lax.Precision.HIGHEST
    h = jnp.dot(mean, w_ref[0:D, :], preferred_element_type=jnp.float32,
                precision=hp)
    h = h + jnp.dot(mx, w_ref[D:2 * D, :], preferred_element_type=jnp.float32,
                    precision=hp)
    h = h + jnp.dot(sq, w_ref[2 * D:3 * D, :],
                    preferred_element_type=jnp.float32, precision=hp)
    out_ref[...] = xe_ref[...] + jnp.maximum(h + b_ref[...], 0.0)


_edge_call = pl.pallas_call(
    _edge_body,
    grid=(EGRID,),
    in_specs=[
        pl.BlockSpec((EBLK, D), lambda i: (i, 0)),
        pl.BlockSpec((EBLK, D), lambda i: (i, 0)),
        pl.BlockSpec((EBLK, D), lambda i: (i, 0)),
        pl.BlockSpec((3 * D, D), lambda i: (0, 0)),
        pl.BlockSpec((1, D), lambda i: (0, 0)),
    ],
    out_specs=pl.BlockSpec((EBLK, D), lambda i: (i, 0)),
    out_shape=jax.ShapeDtypeStruct((E, D), jnp.float32),
)


def _vert_body(gath_ref, m_ref, sp_ref, ep_ref, xv_ref, w_ref, b_ref, out_ref):
    g = gath_ref[...]
    msk = m_ref[...] > 0.5
    gs = jnp.where(msk, g, 0.0)
    gm = jnp.where(msk, g, -1e30)
    s = jnp.sum(gs.reshape(VB, K, D), axis=1)
    mx = jnp.max(gm.reshape(VB, K, D), axis=1)
    sq = jnp.sum((gs * gs).reshape(VB, K, D), axis=1)
    deg = (ep_ref[...] - sp_ref[...]).astype(jnp.float32)
    inv = 1.0 / jnp.maximum(deg, 1.0)
    mean = s * inv
    mx = jnp.where(deg > 0.0, mx, 0.0)
    sqm = sq * inv
    hp = lax.Precision.HIGHEST
    h = jnp.dot(mean, w_ref[0:D, :], preferred_element_type=jnp.float32,
                precision=hp)
    h = h + jnp.dot(mx, w_ref[D:2 * D, :], preferred_element_type=jnp.float32,
                    precision=hp)
    h = h + jnp.dot(sqm, w_ref[2 * D:3 * D, :],
                    preferred_element_type=jnp.float32, precision=hp)
    out_ref[...] = xv_ref[...] + jnp.maximum(h + b_ref[...], 0.0)


_vert_call = pl.pallas_call(
    _vert_body,
    grid=(VGRID,),
    in_specs=[
        pl.BlockSpec((VB * K, D), lambda i: (i, 0)),
        pl.BlockSpec((VB * K, 1), lambda i: (i, 0)),
        pl.BlockSpec((VB, 1), lambda i: (i, 0)),
        pl.BlockSpec((VB, 1), lambda i: (i, 0)),
        pl.BlockSpec((VB, D), lambda i: (i, 0)),
        pl.BlockSpec((3 * D, D), lambda i: (0, 0)),
        pl.BlockSpec((1, D), lambda i: (0, 0)),
    ],
    out_specs=pl.BlockSpec((VB, D), lambda i: (i, 0)),
    out_shape=jax.ShapeDtypeStruct((NP, D), jnp.float32),
)


def kernel(edge_index, xe, xv, line_W, line_b, linv_W, linv_b):
    ei0 = edge_index[0]
    ei1 = edge_index[1]

    endpoints = jnp.concatenate([ei0, ei1])
    iota2e = jnp.arange(2 * E, dtype=jnp.int32)
    sorted_rows, order = lax.sort((endpoints, iota2e), num_keys=1,
                                  is_stable=False)
    isb = jnp.concatenate(
        [jnp.ones((1,), jnp.bool_), sorted_rows[1:] != sorted_rows[:-1]])
    seg_start = lax.cummax(jnp.where(isb, iota2e, 0))
    off_in_seg = iota2e - seg_start
    posj = jnp.where(off_in_seg < K,
                     sorted_rows * K + off_in_seg, NSLOT).astype(jnp.int32)
    _, scfull = lax.sort((order, posj), num_keys=1, is_stable=False)
    sc0 = scfull[:E]
    sc1 = scfull[E:]
    rowptr = jnp.searchsorted(
        sorted_rows, jnp.arange(NP + 1, dtype=jnp.int32),
        method="sort").astype(jnp.int32)
    sp = rowptr[:NP, None]
    ep = rowptr[1:NP + 1, None]
    deg = ep - sp
    valid = jnp.arange(K, dtype=jnp.int32)[None, :] < deg
    vmask = valid.astype(jnp.float32).reshape(NSLOT, 1)

    xvp = jnp.concatenate([xv, jnp.zeros((NP - N, D), jnp.float32)])
    xeb = xe
    b_line = line_b.reshape(1, D)
    b_linv = linv_b.reshape(1, D)

    for _ in range(2):
        s0, s1 = _sc_gather2(ei0, ei1, xvp)
        xeb = _edge_call(s0, s1, xeb, line_W, b_line)
        gath = _sc_scatter(sc0, sc1, xeb)
        xvp = _vert_call(gath, vmask, sp, ep, xvp, linv_W, b_linv)

    return (xeb, xvp[:N])
